# Initial kernel scaffold; baseline (speedup 1.0000x reference)
import jax
import jax.numpy as jnp
from jax.experimental import pallas as pl


def kernel(x):
    raise NotImplementedError("write your pallas kernel here")


# speedup vs baseline: 2.9300x; 2.9300x over previous
import functools

import jax
import jax.numpy as jnp
from jax import lax
from jax.experimental import pallas as pl
from jax.experimental.pallas import tpu as pltpu
from jax.experimental.pallas import tpu_sc as plsc

K = 64
ROWS = 128
COLS = 32768
L = 16
NVREG = COLS // L
GROUP = 32
NGROUP = NVREG // GROUP
NWORK = 32
ROWS_PER_W = ROWS // NWORK


def _to_mono_u32(v_f32):
  i = lax.bitcast_convert_type(v_f32, jnp.int32)
  s = lax.shift_right_arithmetic(i, jnp.full((L,), 31, jnp.int32))
  flip = lax.bitwise_or(s, jnp.full((L,), jnp.int32(-2147483648)))
  return lax.bitcast_convert_type(lax.bitwise_xor(i, flip), jnp.uint32)


def _from_mono_u32(u):
  i = lax.bitcast_convert_type(u, jnp.int32)
  s = lax.shift_right_arithmetic(i, jnp.full((L,), 31, jnp.int32))
  flip = lax.bitwise_or(
      lax.bitwise_not(s), jnp.full((L,), jnp.int32(-2147483648)))
  return lax.bitcast_convert_type(lax.bitwise_xor(i, flip), jnp.float32)


def _kth_largest(buf, nvregs, k):
  ones = jnp.full((L,), 1, jnp.uint32)

  def bit_body(bi, t_vec):
    b = 31 - bi
    cand = lax.bitwise_or(t_vec, lax.shift_left(ones, jnp.full((L,), b,
                                                              jnp.uint32)))

    def cnt_body(i, acc):
      v = buf[pl.ds(i * L, L)]
      m = v >= cand
      return acc + jnp.where(m, 1, 0).astype(jnp.int32)

    cnt = lax.fori_loop(0, nvregs, cnt_body, jnp.zeros((L,), jnp.int32))
    total = jnp.sum(cnt)
    take = jnp.broadcast_to(total >= k, (L,))
    return jnp.where(take, cand, t_vec)

  return lax.fori_loop(0, 32, bit_body, jnp.zeros((L,), jnp.uint32))


def _row_topk_mean(row_ref, gmax_ref, cand_ref):
  def ga_body(g, _):
    base = g * (GROUP * L)
    acc = jnp.zeros((L,), jnp.uint32)
    for j in range(GROUP):
      acc = jnp.maximum(acc, _to_mono_u32(row_ref[pl.ds(base + j * L, L)]))
    gmax_ref[pl.ds(g * L, L)] = acc
    return 0

  lax.fori_loop(0, NGROUP, ga_body, 0)

  big_t = _kth_largest(gmax_ref, NGROUP, K)

  def cb_body(i, off):
    u = _to_mono_u32(row_ref[pl.ds(i * L, L)])
    m = u >= big_t
    any_hit = jnp.any(m)

    @pl.when(any_hit)
    def _():
      cand_ref[pl.ds(off, L)] = jnp.where(m, u, jnp.zeros((L,), jnp.uint32))

    return jnp.where(any_hit, off + L, off)

  off = lax.fori_loop(0, NVREG, cb_body, jnp.int32(0))
  ncand_vregs = off // L

  t_vec = _kth_largest(cand_ref, ncand_vregs, K)

  def fin_body(i, carry):
    s, c = carry
    u = cand_ref[pl.ds(i * L, L)]
    m = u > t_vec
    s = s + jnp.where(m, _from_mono_u32(u), jnp.zeros((L,), jnp.float32))
    c = c + jnp.where(m, 1, 0).astype(jnp.int32)
    return s, c

  s_vec, c_vec = lax.fori_loop(
      0, ncand_vregs, fin_body,
      (jnp.zeros((L,), jnp.float32), jnp.zeros((L,), jnp.int32)))
  sum_gt = jnp.sum(s_vec)
  n_gt = jnp.sum(c_vec)
  t_val = jnp.max(_from_mono_u32(t_vec))
  total = sum_gt + (jnp.float32(K) - n_gt.astype(jnp.float32)) * t_val
  return total * jnp.float32(1.0 / K)


def _sc_kernel(x_hbm, out_hbm, row_ref, gmax_ref, cand_ref, res_ref):
  wid = lax.axis_index("s") * 2 + lax.axis_index("c")
  lanes = lax.iota(jnp.int32, L)
  res = jnp.zeros((L,), jnp.float32)
  for r in range(ROWS_PER_W):
    row = wid * ROWS_PER_W + r
    pltpu.sync_copy(x_hbm.at[row], row_ref)
    mean_r = _row_topk_mean(row_ref, gmax_ref, cand_ref)
    res = jnp.where(lanes == r, jnp.broadcast_to(mean_r, (L,)), res)
  res_ref[...] = res
  pltpu.sync_copy(res_ref, out_hbm.at[wid])


@jax.jit
def kernel(x):
  mesh = plsc.VectorSubcoreMesh(
      core_axis_name="c", subcore_axis_name="s", num_cores=2, num_subcores=16)
  out2d = pl.kernel(
      _sc_kernel,
      out_type=jax.ShapeDtypeStruct((NWORK, L), jnp.float32),
      mesh=mesh,
      compiler_params=pltpu.CompilerParams(needs_layout_passes=False),
      scratch_types=[
          pltpu.VMEM((COLS,), jnp.float32),
          pltpu.VMEM((NGROUP * L,), jnp.uint32),
          pltpu.VMEM((COLS,), jnp.uint32),
          pltpu.VMEM((L,), jnp.float32),
      ],
  )(x)
  return out2d[:, :ROWS_PER_W].reshape(ROWS)


# speedup vs baseline: 3.6651x; 1.2509x over previous
import functools

import jax
import jax.numpy as jnp
from jax import lax
from jax.experimental import pallas as pl
from jax.experimental.pallas import tpu as pltpu
from jax.experimental.pallas import tpu_sc as plsc

K = 64
ROWS = 128
COLS = 32768
L = 16
NVREG = COLS // L
GROUP = 32
NGROUP = NVREG // GROUP
NWORK = 32
ROWS_PER_W = ROWS // NWORK


def _to_mono_u32(v_f32):
  i = lax.bitcast_convert_type(v_f32, jnp.int32)
  s = lax.shift_right_arithmetic(i, jnp.full((L,), 31, jnp.int32))
  flip = lax.bitwise_or(s, jnp.full((L,), jnp.int32(-2147483648)))
  return lax.bitcast_convert_type(lax.bitwise_xor(i, flip), jnp.uint32)


def _from_mono_u32(u):
  i = lax.bitcast_convert_type(u, jnp.int32)
  s = lax.shift_right_arithmetic(i, jnp.full((L,), 31, jnp.int32))
  flip = lax.bitwise_or(
      lax.bitwise_not(s), jnp.full((L,), jnp.int32(-2147483648)))
  return lax.bitcast_convert_type(lax.bitwise_xor(i, flip), jnp.float32)


def _kth_largest(buf, nblk4, k):
  ones = jnp.full((L,), 1, jnp.uint32)

  def bit_body(bi, t_vec):
    b = 31 - bi
    cand = lax.bitwise_or(t_vec, lax.shift_left(ones, jnp.full((L,), b,
                                                              jnp.uint32)))

    def cnt_body(i, acc):
      for j in range(4):
        v = buf[pl.ds((i * 4 + j) * L, L)]
        acc = acc + jnp.where(v >= cand, 1, 0).astype(jnp.int32)
      return acc

    cnt = lax.fori_loop(0, nblk4, cnt_body, jnp.zeros((L,), jnp.int32))
    total = jnp.sum(cnt)
    take = jnp.broadcast_to(total >= k, (L,))
    return jnp.where(take, cand, t_vec)

  return lax.fori_loop(0, 32, bit_body, jnp.zeros((L,), jnp.uint32))


def _row_topk_mean(row_ref, gmax_ref, cand_ref):
  neg_inf = jnp.full((L,), -jnp.inf, jnp.float32)

  def ga_body(g, _):
    base = g * (GROUP * L)
    acc = neg_inf
    for j in range(GROUP):
      acc = jnp.maximum(acc, row_ref[pl.ds(base + j * L, L)])
    gmax_ref[pl.ds(g * L, L)] = _to_mono_u32(acc)
    return 0

  lax.fori_loop(0, NGROUP, ga_body, 0)

  big_t = _kth_largest(gmax_ref, NGROUP // 4, K)
  big_t_f = _from_mono_u32(big_t)

  def cb_body(i, off):
    v = row_ref[pl.ds(i * L, L)]
    m = v >= big_t_f
    any_hit = jnp.any(m)

    @pl.when(any_hit)
    def _():
      u = _to_mono_u32(v)
      cand_ref[pl.ds(off, L)] = jnp.where(m, u, jnp.zeros((L,), jnp.uint32))

    return jnp.where(any_hit, off + L, off)

  off = lax.fori_loop(0, NVREG, cb_body, jnp.int32(0))

  zero_u = jnp.zeros((L,), jnp.uint32)
  for j in range(3):
    cand_ref[pl.ds(off + j * L, L)] = zero_u
  ncand_vregs = (off // L + 3) // 4 * 4

  t_vec = _kth_largest(cand_ref, ncand_vregs // 4, K)

  def fin_body(i, carry):
    s, c = carry
    for j in range(4):
      u = cand_ref[pl.ds((i * 4 + j) * L, L)]
      m = u > t_vec
      s = s + jnp.where(m, _from_mono_u32(u), jnp.zeros((L,), jnp.float32))
      c = c + jnp.where(m, 1, 0).astype(jnp.int32)
    return s, c

  s_vec, c_vec = lax.fori_loop(
      0, ncand_vregs // 4, fin_body,
      (jnp.zeros((L,), jnp.float32), jnp.zeros((L,), jnp.int32)))
  sum_gt = jnp.sum(s_vec)
  n_gt = jnp.sum(c_vec)
  t_val = jnp.max(_from_mono_u32(t_vec))
  total = sum_gt + (jnp.float32(K) - n_gt.astype(jnp.float32)) * t_val
  return total * jnp.float32(1.0 / K)


def _sc_kernel(x_hbm, out_hbm, row_ref, gmax_ref, cand_ref, res_ref):
  wid = lax.axis_index("s") * 2 + lax.axis_index("c")
  lanes = lax.iota(jnp.int32, L)
  res = jnp.zeros((L,), jnp.float32)
  for r in range(ROWS_PER_W):
    row = wid * ROWS_PER_W + r
    pltpu.sync_copy(x_hbm.at[row], row_ref)
    mean_r = _row_topk_mean(row_ref, gmax_ref, cand_ref)
    res = jnp.where(lanes == r, jnp.broadcast_to(mean_r, (L,)), res)
  res_ref[...] = res
  pltpu.sync_copy(res_ref, out_hbm.at[wid])


@jax.jit
def kernel(x):
  mesh = plsc.VectorSubcoreMesh(
      core_axis_name="c", subcore_axis_name="s", num_cores=2, num_subcores=16)
  out2d = pl.kernel(
      _sc_kernel,
      out_type=jax.ShapeDtypeStruct((NWORK, L), jnp.float32),
      mesh=mesh,
      compiler_params=pltpu.CompilerParams(needs_layout_passes=False),
      scratch_types=[
          pltpu.VMEM((COLS,), jnp.float32),
          pltpu.VMEM((NGROUP * L,), jnp.uint32),
          pltpu.VMEM((COLS + 3 * L,), jnp.uint32),
          pltpu.VMEM((L,), jnp.float32),
      ],
  )(x)
  return out2d[:, :ROWS_PER_W].reshape(ROWS)
